# baseline (speedup 1.0000x reference)
import functools

import jax
import jax.numpy as jnp
from jax import lax
from jax.experimental import pallas as pl
from jax.experimental.pallas import tpu as pltpu
from jax.experimental.pallas import tpu_sc as plsc

N = 10000
T = 64
E = 640000
D = 64
G = 1024

NPAD = 10240
GPAD = 2048
NC = 2
NS = 16
NW = NC * NS
CHUNK = 128


def _even(k):
    return k + (k % 2)


KE = _even((E + NW * CHUNK - 1) // (NW * CHUNK))
EPAD = NW * CHUNK * KE
KP = _even((N + NW * CHUNK - 1) // (NW * CHUNK))
PPAD = NW * CHUNK * KP

_mesh = plsc.VectorSubcoreMesh(core_axis_name="c", subcore_axis_name="s")


def _zero_vec(ref, n):
    z = jnp.zeros((16,), jnp.float32)
    for i in range(n // 16):
        ref[pl.ds(i * 16, 16)] = z


def _zero_rows(ref):
    z = jnp.zeros((16,), jnp.float32)
    for r in range(16):
        for c in range(4):
            ref[r, pl.ds(c * 16, 16)] = z


@functools.partial(
    pl.kernel,
    out_type=(
        jax.ShapeDtypeStruct((NC, NPAD), jnp.float32),
        jax.ShapeDtypeStruct((NC, GPAD), jnp.float32),
    ),
    mesh=_mesh,
    scratch_types=[
        pltpu.VMEM((KE, CHUNK), jnp.int32),
        pltpu.VMEM((KP, CHUNK), jnp.int32),
        pltpu.VMEM((CHUNK,), jnp.float32),
        pltpu.VMEM((CHUNK,), jnp.float32),
        pltpu.VMEM_SHARED((NPAD,), jnp.float32),
        pltpu.VMEM_SHARED((GPAD,), jnp.float32),
    ],
    compiler_params=pltpu.CompilerParams(use_tc_tiling_on_sc=False),
)
def _sc_counts(dst_hbm, bat_hbm, deg_out, cnt_out, dst_v, bat_v, ones_v, z_v,
               dacc, cacc):
    cid = lax.axis_index("c")
    sid = lax.axis_index("s")
    wid = sid * NC + cid
    one = jnp.full((16,), 1.0, jnp.float32)
    for i in range(CHUNK // 16):
        ones_v[pl.ds(i * 16, 16)] = one
    _zero_vec(z_v, CHUNK)
    for j in range(NPAD // NS // CHUNK):
        pltpu.sync_copy(z_v, dacc.at[pl.ds(sid * (NPAD // NS) + j * CHUNK, CHUNK)])
    pltpu.sync_copy(z_v, cacc.at[pl.ds(sid * (GPAD // NS), GPAD // NS)])
    pltpu.sync_copy(dst_hbm.at[wid], dst_v)
    pltpu.sync_copy(bat_hbm.at[wid], bat_v)
    plsc.subcore_barrier()

    def body(i, _):
        pltpu.sync_copy(ones_v, dacc.at[dst_v.at[i]], add=True)
        return 0

    lax.fori_loop(0, KE, body, 0)
    for i in range(KP):
        pltpu.sync_copy(ones_v, cacc.at[bat_v.at[i]], add=True)
    plsc.subcore_barrier()
    pltpu.sync_copy(dacc.at[pl.ds(sid * (NPAD // NS), NPAD // NS)],
                    deg_out.at[cid, pl.ds(sid * (NPAD // NS), NPAD // NS)])
    pltpu.sync_copy(cacc.at[pl.ds(sid * (GPAD // NS), GPAD // NS)],
                    cnt_out.at[cid, pl.ds(sid * (GPAD // NS), GPAD // NS)])


def _make_sc_scatter(K, A):
    @functools.partial(
        pl.kernel,
        out_type=jax.ShapeDtypeStruct((NC, A, D), jnp.float32),
        mesh=_mesh,
        scratch_types=[
            pltpu.VMEM((K, CHUNK), jnp.int32),
            pltpu.VMEM((K, CHUNK), jnp.int32),
            pltpu.VMEM((CHUNK, D), jnp.float32),
            pltpu.VMEM((CHUNK, D), jnp.float32),
            pltpu.VMEM((16, D), jnp.float32),
            pltpu.VMEM_SHARED((A, D), jnp.float32),
            pltpu.SemaphoreType.DMA,
            pltpu.SemaphoreType.DMA,
        ],
        compiler_params=pltpu.CompilerParams(use_tc_tiling_on_sc=False),
    )
    def sc_scatter(y_hbm, src_hbm, dst_hbm, out_hbm, src_v, dst_v, rows_a,
                   rows_b, zrow, acc, sem_a, sem_b):
        cid = lax.axis_index("c")
        sid = lax.axis_index("s")
        wid = sid * NC + cid
        rpt = A // NS
        _zero_rows(zrow)
        for j in range(rpt // 16):
            pltpu.sync_copy(zrow, acc.at[pl.ds(sid * rpt + j * 16, 16), :])
        pltpu.sync_copy(src_hbm.at[wid], src_v)
        pltpu.sync_copy(dst_hbm.at[wid], dst_v)
        plsc.subcore_barrier()

        pltpu.async_copy(y_hbm.at[src_v.at[0]], rows_a, sem_a)
        pltpu.async_copy(y_hbm.at[src_v.at[1]], rows_b, sem_b)

        def body(j, _):
            i = 2 * j
            pltpu.make_async_copy(y_hbm.at[src_v.at[i]], rows_a, sem_a).wait()
            pltpu.sync_copy(rows_a, acc.at[dst_v.at[i]], add=True)
            pltpu.async_copy(y_hbm.at[src_v.at[i + 2]], rows_a, sem_a)
            pltpu.make_async_copy(y_hbm.at[src_v.at[i + 1]], rows_b, sem_b).wait()
            pltpu.sync_copy(rows_b, acc.at[dst_v.at[i + 1]], add=True)
            pltpu.async_copy(y_hbm.at[src_v.at[i + 3]], rows_b, sem_b)
            return 0

        lax.fori_loop(0, K // 2 - 1, body, 0)
        pltpu.make_async_copy(y_hbm.at[src_v.at[K - 2]], rows_a, sem_a).wait()
        pltpu.sync_copy(rows_a, acc.at[dst_v.at[K - 2]], add=True)
        pltpu.make_async_copy(y_hbm.at[src_v.at[K - 1]], rows_b, sem_b).wait()
        pltpu.sync_copy(rows_b, acc.at[dst_v.at[K - 1]], add=True)
        plsc.subcore_barrier()
        pltpu.sync_copy(acc.at[pl.ds(sid * rpt, rpt), :],
                        out_hbm.at[cid, pl.ds(sid * rpt, rpt), :])

    return sc_scatter


_sc_scatter_edges = _make_sc_scatter(KE, NPAD)
_sc_scatter_pool = _make_sc_scatter(KP, GPAD)


MBLK = 1280


MF = MBLK // 2


def _gru_body(x_ref, wr2_ref, wz2_ref, wn2_ref, mr_ref, mz_ref, mn_ref,
              br_ref, bz_ref, bin_ref, bhn_ref, w1t2_ref, dinv_ref, y_ref):
    x = x_ref[...]
    wr2 = wr2_ref[...]
    wz2 = wz2_ref[...]
    wn2 = wn2_ref[...]
    mr = mr_ref[...]
    mz = mz_ref[...]
    mn = mn_ref[...]
    br = br_ref[...]
    bz = bz_ref[...]
    bin_ = bin_ref[...]
    bhn = bhn_ref[...]
    lane64 = lax.broadcasted_iota(jnp.int32, (1, 2 * D), 1) & (D - 1)

    def step(t, h):
        s = jnp.where(lane64 == t, x, 0.0)
        r = jax.nn.sigmoid(jnp.dot(s, mr, preferred_element_type=jnp.float32)
                           + jnp.dot(h, wr2, preferred_element_type=jnp.float32) + br)
        z = jax.nn.sigmoid(jnp.dot(s, mz, preferred_element_type=jnp.float32)
                           + jnp.dot(h, wz2, preferred_element_type=jnp.float32) + bz)
        n = jnp.tanh(jnp.dot(s, mn, preferred_element_type=jnp.float32) + bin_
                     + r * (jnp.dot(h, wn2, preferred_element_type=jnp.float32) + bhn))
        return (1.0 - z) * n + z * h

    h = lax.fori_loop(0, T, step, jnp.zeros((MF, 2 * D), jnp.float32))
    y_ref[...] = jnp.dot(h, w1t2_ref[...], preferred_element_type=jnp.float32) * dinv_ref[...]


def _bdiag(w):
    z = jnp.zeros((2 * D, 2 * D), jnp.float32)
    return z.at[0:D, 0:D].set(w).at[D:2 * D, D:2 * D].set(w)


def _run_gru(x_pad, w_hh, w_ih, b_ih, b_hh, w1t, dinv):
    wr = jnp.transpose(w_hh[0:D])
    wz = jnp.transpose(w_hh[D:2 * D])
    wn = jnp.transpose(w_hh[2 * D:3 * D])
    ones = jnp.ones((D, 1), jnp.float32)
    mr = _bdiag(ones * w_ih[0:D, 0][None, :])
    mz = _bdiag(ones * w_ih[D:2 * D, 0][None, :])
    mn = _bdiag(ones * w_ih[2 * D:3 * D, 0][None, :])
    wr2, wz2, wn2 = _bdiag(wr), _bdiag(wz), _bdiag(wn)
    two = lambda b: jnp.concatenate([b, b])[None, :]
    br = two(b_ih[0:D] + b_hh[0:D])
    bz = two(b_ih[D:2 * D] + b_hh[D:2 * D])
    bin_ = two(b_ih[2 * D:3 * D])
    bhn = two(b_hh[2 * D:3 * D])
    w1t2 = _bdiag(w1t)
    x_f = x_pad.reshape(NPAD // 2, 2 * D)
    dinv_f = jnp.broadcast_to(dinv.reshape(NPAD // 2, 2, 1),
                              (NPAD // 2, 2, D)).reshape(NPAD // 2, 2 * D)
    full = lambda s: pl.BlockSpec(s, lambda i: (0,) * len(s))
    y_f = pl.pallas_call(
        _gru_body,
        out_shape=jax.ShapeDtypeStruct((NPAD // 2, 2 * D), jnp.float32),
        grid=(NPAD // MBLK,),
        in_specs=[
            pl.BlockSpec((MF, 2 * D), lambda i: (i, 0)),
            full((2 * D, 2 * D)), full((2 * D, 2 * D)), full((2 * D, 2 * D)),
            full((2 * D, 2 * D)), full((2 * D, 2 * D)), full((2 * D, 2 * D)),
            full((1, 2 * D)), full((1, 2 * D)), full((1, 2 * D)), full((1, 2 * D)),
            full((2 * D, 2 * D)),
            pl.BlockSpec((MF, 2 * D), lambda i: (i, 0)),
        ],
        out_specs=pl.BlockSpec((MF, 2 * D), lambda i: (i, 0)),
    )(x_f, wr2, wz2, wn2, mr, mz, mn, br, bz, bin_, bhn, w1t2, dinv_f)
    return y_f.reshape(NPAD, D)


def _mid_body(y_ref, p_ref, dinv_ref, b_ref, wt_ref, o_ref):
    agg = y_ref[...] + p_ref[0] + p_ref[1]
    h = jax.nn.relu(agg * dinv_ref[...] + b_ref[...])
    o_ref[...] = jnp.dot(h, wt_ref[...], preferred_element_type=jnp.float32) * dinv_ref[...]


def _final_body(y_ref, p_ref, dinv_ref, b_ref, o_ref):
    agg = y_ref[...] + p_ref[0] + p_ref[1]
    o_ref[...] = jax.nn.relu(agg * dinv_ref[...] + b_ref[...])


def _run_mid(y, parts, dinv, b, wt):
    full = lambda s: pl.BlockSpec(s, lambda i: (0,) * len(s))
    return pl.pallas_call(
        _mid_body,
        out_shape=jax.ShapeDtypeStruct((NPAD, D), jnp.float32),
        grid=(NPAD // MBLK,),
        in_specs=[
            pl.BlockSpec((MBLK, D), lambda i: (i, 0)),
            pl.BlockSpec((NC, MBLK, D), lambda i: (0, i, 0)),
            pl.BlockSpec((MBLK, 1), lambda i: (i, 0)),
            full((1, D)), full((D, D)),
        ],
        out_specs=pl.BlockSpec((MBLK, D), lambda i: (i, 0)),
    )(y, parts, dinv, b, wt)


def _run_final(y, parts, dinv, b):
    full = lambda s: pl.BlockSpec(s, lambda i: (0,) * len(s))
    return pl.pallas_call(
        _final_body,
        out_shape=jax.ShapeDtypeStruct((NPAD, D), jnp.float32),
        grid=(NPAD // MBLK,),
        in_specs=[
            pl.BlockSpec((MBLK, D), lambda i: (i, 0)),
            pl.BlockSpec((NC, MBLK, D), lambda i: (0, i, 0)),
            pl.BlockSpec((MBLK, 1), lambda i: (i, 0)),
            full((1, D)),
        ],
        out_specs=pl.BlockSpec((MBLK, D), lambda i: (i, 0)),
    )(y, parts, dinv, b)


def _pool_body(pp_ref, cnt_ref, fcwt_ref, fcb_ref, o_ref):
    s = pp_ref[0, 0:G, :] + pp_ref[1, 0:G, :]
    pooled = s / jnp.maximum(cnt_ref[...], 1.0)
    o_ref[...] = jnp.dot(pooled, fcwt_ref[...], preferred_element_type=jnp.float32) + fcb_ref[...]


def _run_pool(pool_parts, cnt, fc_w, fc_b):
    fcwt = jnp.zeros((D, 128), jnp.float32).at[:, 0:10].set(jnp.transpose(fc_w))
    fcb = jnp.zeros((1, 128), jnp.float32).at[0, 0:10].set(fc_b)
    out = pl.pallas_call(
        _pool_body,
        out_shape=jax.ShapeDtypeStruct((G, 128), jnp.float32),
    )(pool_parts, cnt, fcwt, fcb)
    return out[:, 0:10]


def kernel(x, edge_index, batch, w_ih, w_hh, b_ih, b_hh, w1, b1, w2, b2,
           fc_w, fc_b):
    epad_junk = N + (jnp.arange(EPAD - E, dtype=jnp.int32) % (NPAD - N))
    src = jnp.concatenate([edge_index[0], epad_junk])
    dst = jnp.concatenate([edge_index[1], epad_junk])
    src3 = src.reshape(NW, KE, CHUNK)
    dst3 = dst.reshape(NW, KE, CHUNK)
    ppad_junk = jnp.arange(PPAD - N, dtype=jnp.int32)
    psrc = jnp.concatenate([jnp.arange(N, dtype=jnp.int32),
                            N + (ppad_junk % (NPAD - N))]).reshape(NW, KP, CHUNK)
    pdst = jnp.concatenate([batch, G + (ppad_junk % (GPAD - G))]).reshape(NW, KP, CHUNK)

    deg_p, cnt_p = _sc_counts(dst3, pdst)
    deg = deg_p[0] + deg_p[1] + 1.0
    dinv = lax.rsqrt(deg)[:, None]
    cnt = (cnt_p[0] + cnt_p[1])[0:G, None]

    x_pad = jnp.zeros((NPAD, T), jnp.float32).at[0:N].set(x)
    w1t = jnp.transpose(w1)
    y1 = _run_gru(x_pad, w_hh, w_ih, b_ih, b_hh, w1t, dinv)

    parts1 = _sc_scatter_edges(y1, src3, dst3)
    y2 = _run_mid(y1, parts1, dinv, b1[None, :], jnp.transpose(w2))

    parts2 = _sc_scatter_edges(y2, src3, dst3)
    h2 = _run_final(y2, parts2, dinv, b2[None, :])

    pool_parts = _sc_scatter_pool(h2, psrc, pdst)
    return _run_pool(pool_parts, cnt, fc_w, fc_b)


# speedup vs baseline: 1.0570x; 1.0570x over previous
import functools

import jax
import jax.numpy as jnp
from jax import lax
from jax.experimental import pallas as pl
from jax.experimental.pallas import tpu as pltpu
from jax.experimental.pallas import tpu_sc as plsc

N = 10000
T = 64
E = 640000
D = 64
G = 1024

NPAD = 10240
GPAD = 2048
NC = 2
NS = 16
NW = NC * NS
CHUNK = 128


def _even(k):
    return k + (k % 2)


KE = _even((E + NW * CHUNK - 1) // (NW * CHUNK))
EPAD = NW * CHUNK * KE
KP = _even((N + NW * CHUNK - 1) // (NW * CHUNK))
PPAD = NW * CHUNK * KP

_mesh = plsc.VectorSubcoreMesh(core_axis_name="c", subcore_axis_name="s")


def _zero_vec(ref, n):
    z = jnp.zeros((16,), jnp.float32)
    for i in range(n // 16):
        ref[pl.ds(i * 16, 16)] = z


def _zero_rows(ref):
    z = jnp.zeros((16,), jnp.float32)
    for r in range(16):
        for c in range(4):
            ref[r, pl.ds(c * 16, 16)] = z


@functools.partial(
    pl.kernel,
    out_type=(
        jax.ShapeDtypeStruct((NC, NPAD), jnp.float32),
        jax.ShapeDtypeStruct((NC, GPAD), jnp.float32),
    ),
    mesh=_mesh,
    scratch_types=[
        pltpu.VMEM((KE, CHUNK), jnp.int32),
        pltpu.VMEM((KP, CHUNK), jnp.int32),
        pltpu.VMEM((CHUNK,), jnp.float32),
        pltpu.VMEM((CHUNK,), jnp.float32),
        pltpu.VMEM_SHARED((NPAD,), jnp.float32),
        pltpu.VMEM_SHARED((GPAD,), jnp.float32),
    ],
    compiler_params=pltpu.CompilerParams(use_tc_tiling_on_sc=False),
)
def _sc_counts(dst_hbm, bat_hbm, deg_out, cnt_out, dst_v, bat_v, ones_v, z_v,
               dacc, cacc):
    cid = lax.axis_index("c")
    sid = lax.axis_index("s")
    wid = sid * NC + cid
    one = jnp.full((16,), 1.0, jnp.float32)
    for i in range(CHUNK // 16):
        ones_v[pl.ds(i * 16, 16)] = one
    _zero_vec(z_v, CHUNK)
    for j in range(NPAD // NS // CHUNK):
        pltpu.sync_copy(z_v, dacc.at[pl.ds(sid * (NPAD // NS) + j * CHUNK, CHUNK)])
    pltpu.sync_copy(z_v, cacc.at[pl.ds(sid * (GPAD // NS), GPAD // NS)])
    pltpu.sync_copy(dst_hbm.at[wid], dst_v)
    pltpu.sync_copy(bat_hbm.at[wid], bat_v)
    plsc.subcore_barrier()

    def body(i, _):
        pltpu.sync_copy(ones_v, dacc.at[dst_v.at[i]], add=True)
        return 0

    lax.fori_loop(0, KE, body, 0)
    for i in range(KP):
        pltpu.sync_copy(ones_v, cacc.at[bat_v.at[i]], add=True)
    plsc.subcore_barrier()
    pltpu.sync_copy(dacc.at[pl.ds(sid * (NPAD // NS), NPAD // NS)],
                    deg_out.at[cid, pl.ds(sid * (NPAD // NS), NPAD // NS)])
    pltpu.sync_copy(cacc.at[pl.ds(sid * (GPAD // NS), GPAD // NS)],
                    cnt_out.at[cid, pl.ds(sid * (GPAD // NS), GPAD // NS)])


def _make_sc_scatter(K, A):
    @functools.partial(
        pl.kernel,
        out_type=jax.ShapeDtypeStruct((NC, A, D), jnp.float32),
        mesh=_mesh,
        scratch_types=[
            pltpu.VMEM((K, CHUNK), jnp.int32),
            pltpu.VMEM((K, CHUNK), jnp.int32),
            pltpu.VMEM((CHUNK, D), jnp.float32),
            pltpu.VMEM((CHUNK, D), jnp.float32),
            pltpu.VMEM((16, D), jnp.float32),
            pltpu.VMEM_SHARED((A, D), jnp.float32),
            pltpu.SemaphoreType.DMA,
            pltpu.SemaphoreType.DMA,
        ],
        compiler_params=pltpu.CompilerParams(use_tc_tiling_on_sc=False),
    )
    def sc_scatter(y_hbm, src_hbm, dst_hbm, out_hbm, src_v, dst_v, rows_a,
                   rows_b, zrow, acc, sem_a, sem_b):
        cid = lax.axis_index("c")
        sid = lax.axis_index("s")
        wid = sid * NC + cid
        rpt = A // NS
        _zero_rows(zrow)
        for j in range(rpt // 16):
            pltpu.sync_copy(zrow, acc.at[pl.ds(sid * rpt + j * 16, 16), :])
        pltpu.sync_copy(src_hbm.at[wid], src_v)
        pltpu.sync_copy(dst_hbm.at[wid], dst_v)
        plsc.subcore_barrier()

        pltpu.async_copy(y_hbm.at[src_v.at[0]], rows_a, sem_a)
        pltpu.async_copy(y_hbm.at[src_v.at[1]], rows_b, sem_b)

        def body(j, _):
            i = 2 * j
            pltpu.make_async_copy(y_hbm.at[src_v.at[i]], rows_a, sem_a).wait()
            pltpu.sync_copy(rows_a, acc.at[dst_v.at[i]], add=True)
            pltpu.async_copy(y_hbm.at[src_v.at[i + 2]], rows_a, sem_a)
            pltpu.make_async_copy(y_hbm.at[src_v.at[i + 1]], rows_b, sem_b).wait()
            pltpu.sync_copy(rows_b, acc.at[dst_v.at[i + 1]], add=True)
            pltpu.async_copy(y_hbm.at[src_v.at[i + 3]], rows_b, sem_b)
            return 0

        lax.fori_loop(0, K // 2 - 1, body, 0)
        pltpu.make_async_copy(y_hbm.at[src_v.at[K - 2]], rows_a, sem_a).wait()
        pltpu.sync_copy(rows_a, acc.at[dst_v.at[K - 2]], add=True)
        pltpu.make_async_copy(y_hbm.at[src_v.at[K - 1]], rows_b, sem_b).wait()
        pltpu.sync_copy(rows_b, acc.at[dst_v.at[K - 1]], add=True)
        plsc.subcore_barrier()
        pltpu.sync_copy(acc.at[pl.ds(sid * rpt, rpt), :],
                        out_hbm.at[cid, pl.ds(sid * rpt, rpt), :])

    return sc_scatter


_sc_scatter_edges = _make_sc_scatter(KE, NPAD)
_sc_scatter_pool = _make_sc_scatter(KP, GPAD)


MBLK = 1280


MF = MBLK // 2


def _gru_body(x_ref, wr2_ref, wz2_ref, wn2_ref, mr_ref, mz_ref, mn_ref,
              br_ref, bz_ref, bin_ref, bhn_ref, w1t2_ref, dinv_ref, y_ref):
    x = x_ref[...]
    wr2 = wr2_ref[...]
    wz2 = wz2_ref[...]
    wn2 = wn2_ref[...]
    mr = mr_ref[...]
    mz = mz_ref[...]
    mn = mn_ref[...]
    br = br_ref[...]
    bz = bz_ref[...]
    bin_ = bin_ref[...]
    bhn = bhn_ref[...]
    lane64 = lax.broadcasted_iota(jnp.int32, (1, 2 * D), 1) & (D - 1)

    def step(t, h):
        s = jnp.where(lane64 == t, x, 0.0)
        hb = h.astype(jnp.bfloat16)
        r = jax.nn.sigmoid(jnp.dot(s, mr, preferred_element_type=jnp.float32)
                           + jnp.dot(hb, wr2, preferred_element_type=jnp.float32) + br)
        z = jax.nn.sigmoid(jnp.dot(s, mz, preferred_element_type=jnp.float32)
                           + jnp.dot(hb, wz2, preferred_element_type=jnp.float32) + bz)
        n = jnp.tanh(jnp.dot(s, mn, preferred_element_type=jnp.float32) + bin_
                     + r * (jnp.dot(hb, wn2, preferred_element_type=jnp.float32) + bhn))
        return (1.0 - z) * n + z * h

    h = lax.fori_loop(0, T, step, jnp.zeros((MF, 2 * D), jnp.float32))
    y_ref[...] = jnp.dot(h, w1t2_ref[...], preferred_element_type=jnp.float32) * dinv_ref[...]


def _bdiag(w):
    z = jnp.zeros((2 * D, 2 * D), jnp.float32)
    return z.at[0:D, 0:D].set(w).at[D:2 * D, D:2 * D].set(w)


def _run_gru(x_pad, w_hh, w_ih, b_ih, b_hh, w1t, dinv):
    wr = jnp.transpose(w_hh[0:D])
    wz = jnp.transpose(w_hh[D:2 * D])
    wn = jnp.transpose(w_hh[2 * D:3 * D])
    ones = jnp.ones((D, 1), jnp.float32)
    mr = _bdiag(ones * w_ih[0:D, 0][None, :])
    mz = _bdiag(ones * w_ih[D:2 * D, 0][None, :])
    mn = _bdiag(ones * w_ih[2 * D:3 * D, 0][None, :])
    wr2 = _bdiag(wr).astype(jnp.bfloat16)
    wz2 = _bdiag(wz).astype(jnp.bfloat16)
    wn2 = _bdiag(wn).astype(jnp.bfloat16)
    two = lambda b: jnp.concatenate([b, b])[None, :]
    br = two(b_ih[0:D] + b_hh[0:D])
    bz = two(b_ih[D:2 * D] + b_hh[D:2 * D])
    bin_ = two(b_ih[2 * D:3 * D])
    bhn = two(b_hh[2 * D:3 * D])
    w1t2 = _bdiag(w1t)
    x_f = x_pad.reshape(NPAD // 2, 2 * D)
    dinv_f = jnp.broadcast_to(dinv.reshape(NPAD // 2, 2, 1),
                              (NPAD // 2, 2, D)).reshape(NPAD // 2, 2 * D)
    full = lambda s: pl.BlockSpec(s, lambda i: (0,) * len(s))
    y_f = pl.pallas_call(
        _gru_body,
        out_shape=jax.ShapeDtypeStruct((NPAD // 2, 2 * D), jnp.float32),
        grid=(NPAD // MBLK,),
        in_specs=[
            pl.BlockSpec((MF, 2 * D), lambda i: (i, 0)),
            full((2 * D, 2 * D)), full((2 * D, 2 * D)), full((2 * D, 2 * D)),
            full((2 * D, 2 * D)), full((2 * D, 2 * D)), full((2 * D, 2 * D)),
            full((1, 2 * D)), full((1, 2 * D)), full((1, 2 * D)), full((1, 2 * D)),
            full((2 * D, 2 * D)),
            pl.BlockSpec((MF, 2 * D), lambda i: (i, 0)),
        ],
        out_specs=pl.BlockSpec((MF, 2 * D), lambda i: (i, 0)),
    )(x_f, wr2, wz2, wn2, mr, mz, mn, br, bz, bin_, bhn, w1t2, dinv_f)
    return y_f.reshape(NPAD, D)


def _mid_body(y_ref, p_ref, dinv_ref, b_ref, wt_ref, o_ref):
    agg = y_ref[...] + p_ref[0] + p_ref[1]
    h = jax.nn.relu(agg * dinv_ref[...] + b_ref[...])
    o_ref[...] = jnp.dot(h, wt_ref[...], preferred_element_type=jnp.float32) * dinv_ref[...]


def _final_body(y_ref, p_ref, dinv_ref, b_ref, o_ref):
    agg = y_ref[...] + p_ref[0] + p_ref[1]
    o_ref[...] = jax.nn.relu(agg * dinv_ref[...] + b_ref[...])


def _run_mid(y, parts, dinv, b, wt):
    full = lambda s: pl.BlockSpec(s, lambda i: (0,) * len(s))
    return pl.pallas_call(
        _mid_body,
        out_shape=jax.ShapeDtypeStruct((NPAD, D), jnp.float32),
        grid=(NPAD // MBLK,),
        in_specs=[
            pl.BlockSpec((MBLK, D), lambda i: (i, 0)),
            pl.BlockSpec((NC, MBLK, D), lambda i: (0, i, 0)),
            pl.BlockSpec((MBLK, 1), lambda i: (i, 0)),
            full((1, D)), full((D, D)),
        ],
        out_specs=pl.BlockSpec((MBLK, D), lambda i: (i, 0)),
    )(y, parts, dinv, b, wt)


def _run_final(y, parts, dinv, b):
    full = lambda s: pl.BlockSpec(s, lambda i: (0,) * len(s))
    return pl.pallas_call(
        _final_body,
        out_shape=jax.ShapeDtypeStruct((NPAD, D), jnp.float32),
        grid=(NPAD // MBLK,),
        in_specs=[
            pl.BlockSpec((MBLK, D), lambda i: (i, 0)),
            pl.BlockSpec((NC, MBLK, D), lambda i: (0, i, 0)),
            pl.BlockSpec((MBLK, 1), lambda i: (i, 0)),
            full((1, D)),
        ],
        out_specs=pl.BlockSpec((MBLK, D), lambda i: (i, 0)),
    )(y, parts, dinv, b)


def _pool_body(pp_ref, cnt_ref, fcwt_ref, fcb_ref, o_ref):
    s = pp_ref[0, 0:G, :] + pp_ref[1, 0:G, :]
    pooled = s / jnp.maximum(cnt_ref[...], 1.0)
    o_ref[...] = jnp.dot(pooled, fcwt_ref[...], preferred_element_type=jnp.float32) + fcb_ref[...]


def _run_pool(pool_parts, cnt, fc_w, fc_b):
    fcwt = jnp.zeros((D, 128), jnp.float32).at[:, 0:10].set(jnp.transpose(fc_w))
    fcb = jnp.zeros((1, 128), jnp.float32).at[0, 0:10].set(fc_b)
    out = pl.pallas_call(
        _pool_body,
        out_shape=jax.ShapeDtypeStruct((G, 128), jnp.float32),
    )(pool_parts, cnt, fcwt, fcb)
    return out[:, 0:10]


def kernel(x, edge_index, batch, w_ih, w_hh, b_ih, b_hh, w1, b1, w2, b2,
           fc_w, fc_b):
    epad_junk = N + (jnp.arange(EPAD - E, dtype=jnp.int32) % (NPAD - N))
    src = jnp.concatenate([edge_index[0], epad_junk])
    dst = jnp.concatenate([edge_index[1], epad_junk])
    src3 = src.reshape(NW, KE, CHUNK)
    dst3 = dst.reshape(NW, KE, CHUNK)
    ppad_junk = jnp.arange(PPAD - N, dtype=jnp.int32)
    psrc = jnp.concatenate([jnp.arange(N, dtype=jnp.int32),
                            N + (ppad_junk % (NPAD - N))]).reshape(NW, KP, CHUNK)
    pdst = jnp.concatenate([batch, G + (ppad_junk % (GPAD - G))]).reshape(NW, KP, CHUNK)

    deg_p, cnt_p = _sc_counts(dst3, pdst)
    deg = deg_p[0] + deg_p[1] + 1.0
    dinv = lax.rsqrt(deg)[:, None]
    cnt = (cnt_p[0] + cnt_p[1])[0:G, None]

    x_pad = jnp.zeros((NPAD, T), jnp.float32).at[0:N].set(x)
    w1t = jnp.transpose(w1)
    y1 = _run_gru(x_pad, w_hh, w_ih, b_ih, b_hh, w1t, dinv)

    parts1 = _sc_scatter_edges(y1, src3, dst3)
    y2 = _run_mid(y1, parts1, dinv, b1[None, :], jnp.transpose(w2))

    parts2 = _sc_scatter_edges(y2, src3, dst3)
    h2 = _run_final(y2, parts2, dinv, b2[None, :])

    pool_parts = _sc_scatter_pool(h2, psrc, pdst)
    return _run_pool(pool_parts, cnt, fc_w, fc_b)


# speedup vs baseline: 1.0644x; 1.0070x over previous
import functools

import jax
import jax.numpy as jnp
from jax import lax
from jax.experimental import pallas as pl
from jax.experimental.pallas import tpu as pltpu
from jax.experimental.pallas import tpu_sc as plsc

N = 10000
T = 64
E = 640000
D = 64
G = 1024

NPAD = 10240
GPAD = 2048
NC = 2
NS = 16
NW = NC * NS
CHUNK = 128


def _even(k):
    return k + (k % 2)


KE = _even((E + NW * CHUNK - 1) // (NW * CHUNK))
EPAD = NW * CHUNK * KE
KP = _even((N + NW * CHUNK - 1) // (NW * CHUNK))
PPAD = NW * CHUNK * KP

_mesh = plsc.VectorSubcoreMesh(core_axis_name="c", subcore_axis_name="s")


def _zero_vec(ref, n):
    z = jnp.zeros((16,), jnp.float32)
    for i in range(n // 16):
        ref[pl.ds(i * 16, 16)] = z


def _zero_rows(ref):
    z = jnp.zeros((16,), jnp.float32)
    for r in range(16):
        for c in range(4):
            ref[r, pl.ds(c * 16, 16)] = z


@functools.partial(
    pl.kernel,
    out_type=(
        jax.ShapeDtypeStruct((NC, NPAD), jnp.float32),
        jax.ShapeDtypeStruct((NC, GPAD), jnp.float32),
    ),
    mesh=_mesh,
    scratch_types=[
        pltpu.VMEM((KE, CHUNK), jnp.int32),
        pltpu.VMEM((KP, CHUNK), jnp.int32),
        pltpu.VMEM((CHUNK,), jnp.float32),
        pltpu.VMEM((CHUNK,), jnp.float32),
        pltpu.VMEM_SHARED((NPAD,), jnp.float32),
        pltpu.VMEM_SHARED((GPAD,), jnp.float32),
    ],
    compiler_params=pltpu.CompilerParams(use_tc_tiling_on_sc=False),
)
def _sc_counts(dst_hbm, bat_hbm, deg_out, cnt_out, dst_v, bat_v, ones_v, z_v,
               dacc, cacc):
    cid = lax.axis_index("c")
    sid = lax.axis_index("s")
    wid = sid * NC + cid
    one = jnp.full((16,), 1.0, jnp.float32)
    for i in range(CHUNK // 16):
        ones_v[pl.ds(i * 16, 16)] = one
    _zero_vec(z_v, CHUNK)
    for j in range(NPAD // NS // CHUNK):
        pltpu.sync_copy(z_v, dacc.at[pl.ds(sid * (NPAD // NS) + j * CHUNK, CHUNK)])
    pltpu.sync_copy(z_v, cacc.at[pl.ds(sid * (GPAD // NS), GPAD // NS)])
    pltpu.sync_copy(dst_hbm.at[wid], dst_v)
    pltpu.sync_copy(bat_hbm.at[wid], bat_v)
    plsc.subcore_barrier()

    def body(i, _):
        pltpu.sync_copy(ones_v, dacc.at[dst_v.at[i]], add=True)
        return 0

    lax.fori_loop(0, KE, body, 0)
    for i in range(KP):
        pltpu.sync_copy(ones_v, cacc.at[bat_v.at[i]], add=True)
    plsc.subcore_barrier()
    pltpu.sync_copy(dacc.at[pl.ds(sid * (NPAD // NS), NPAD // NS)],
                    deg_out.at[cid, pl.ds(sid * (NPAD // NS), NPAD // NS)])
    pltpu.sync_copy(cacc.at[pl.ds(sid * (GPAD // NS), GPAD // NS)],
                    cnt_out.at[cid, pl.ds(sid * (GPAD // NS), GPAD // NS)])


def _make_sc_scatter(K, A):
    @functools.partial(
        pl.kernel,
        out_type=jax.ShapeDtypeStruct((NC, A, D), jnp.float32),
        mesh=_mesh,
        scratch_types=[
            pltpu.VMEM((K, CHUNK), jnp.int32),
            pltpu.VMEM((K, CHUNK), jnp.int32),
            pltpu.VMEM((CHUNK, D), jnp.float32),
            pltpu.VMEM((CHUNK, D), jnp.float32),
            pltpu.VMEM((16, D), jnp.float32),
            pltpu.VMEM_SHARED((A, D), jnp.float32),
            pltpu.SemaphoreType.DMA,
            pltpu.SemaphoreType.DMA,
        ],
        compiler_params=pltpu.CompilerParams(use_tc_tiling_on_sc=False),
    )
    def sc_scatter(y_hbm, src_hbm, dst_hbm, out_hbm, src_v, dst_v, rows_a,
                   rows_b, zrow, acc, sem_a, sem_b):
        cid = lax.axis_index("c")
        sid = lax.axis_index("s")
        wid = sid * NC + cid
        rpt = A // NS
        _zero_rows(zrow)
        for j in range(rpt // 16):
            pltpu.sync_copy(zrow, acc.at[pl.ds(sid * rpt + j * 16, 16), :])
        pltpu.sync_copy(src_hbm.at[wid], src_v)
        pltpu.sync_copy(dst_hbm.at[wid], dst_v)
        plsc.subcore_barrier()

        pltpu.async_copy(y_hbm.at[src_v.at[0]], rows_a, sem_a)
        pltpu.async_copy(y_hbm.at[src_v.at[1]], rows_b, sem_b)

        def body(j, _):
            i = 2 * j
            pltpu.make_async_copy(y_hbm.at[src_v.at[i]], rows_a, sem_a).wait()
            pltpu.sync_copy(rows_a, acc.at[dst_v.at[i]], add=True)
            pltpu.async_copy(y_hbm.at[src_v.at[i + 2]], rows_a, sem_a)
            pltpu.make_async_copy(y_hbm.at[src_v.at[i + 1]], rows_b, sem_b).wait()
            pltpu.sync_copy(rows_b, acc.at[dst_v.at[i + 1]], add=True)
            pltpu.async_copy(y_hbm.at[src_v.at[i + 3]], rows_b, sem_b)
            return 0

        lax.fori_loop(0, K // 2 - 1, body, 0)
        pltpu.make_async_copy(y_hbm.at[src_v.at[K - 2]], rows_a, sem_a).wait()
        pltpu.sync_copy(rows_a, acc.at[dst_v.at[K - 2]], add=True)
        pltpu.make_async_copy(y_hbm.at[src_v.at[K - 1]], rows_b, sem_b).wait()
        pltpu.sync_copy(rows_b, acc.at[dst_v.at[K - 1]], add=True)
        plsc.subcore_barrier()
        pltpu.sync_copy(acc.at[pl.ds(sid * rpt, rpt), :],
                        out_hbm.at[cid, pl.ds(sid * rpt, rpt), :])

    return sc_scatter


_sc_scatter_edges = _make_sc_scatter(KE, NPAD)
_sc_scatter_pool = _make_sc_scatter(KP, GPAD)


MBLK = 2560


MF = MBLK // 2


def _gru_body(x_ref, wr2_ref, wz2_ref, wn2_ref, mr_ref, mz_ref, mn_ref,
              br_ref, bz_ref, bin_ref, bhn_ref, w1t2_ref, dinv_ref, y_ref):
    x = x_ref[...]
    wr2 = wr2_ref[...]
    wz2 = wz2_ref[...]
    wn2 = wn2_ref[...]
    mr = mr_ref[...]
    mz = mz_ref[...]
    mn = mn_ref[...]
    br = br_ref[...]
    bz = bz_ref[...]
    bin_ = bin_ref[...]
    bhn = bhn_ref[...]
    lane64 = lax.broadcasted_iota(jnp.int32, (1, 2 * D), 1) & (D - 1)
    xb = x.astype(jnp.bfloat16)

    def step(t, h):
        s = jnp.where(lane64 == t, xb, jnp.bfloat16(0.0))
        hb = h.astype(jnp.bfloat16)
        r = jax.nn.sigmoid(jnp.dot(s, mr, preferred_element_type=jnp.float32)
                           + jnp.dot(hb, wr2, preferred_element_type=jnp.float32) + br)
        z = jax.nn.sigmoid(jnp.dot(s, mz, preferred_element_type=jnp.float32)
                           + jnp.dot(hb, wz2, preferred_element_type=jnp.float32) + bz)
        n = jnp.tanh(jnp.dot(s, mn, preferred_element_type=jnp.float32) + bin_
                     + r * (jnp.dot(hb, wn2, preferred_element_type=jnp.float32) + bhn))
        return (1.0 - z) * n + z * h

    h = lax.fori_loop(0, T, step, jnp.zeros((MF, 2 * D), jnp.float32))
    y_ref[...] = jnp.dot(h, w1t2_ref[...], preferred_element_type=jnp.float32) * dinv_ref[...]


def _bdiag(w):
    z = jnp.zeros((2 * D, 2 * D), jnp.float32)
    return z.at[0:D, 0:D].set(w).at[D:2 * D, D:2 * D].set(w)


def _run_gru(x_pad, w_hh, w_ih, b_ih, b_hh, w1t, dinv):
    wr = jnp.transpose(w_hh[0:D])
    wz = jnp.transpose(w_hh[D:2 * D])
    wn = jnp.transpose(w_hh[2 * D:3 * D])
    ones = jnp.ones((D, 1), jnp.float32)
    mr = _bdiag(ones * w_ih[0:D, 0][None, :]).astype(jnp.bfloat16)
    mz = _bdiag(ones * w_ih[D:2 * D, 0][None, :]).astype(jnp.bfloat16)
    mn = _bdiag(ones * w_ih[2 * D:3 * D, 0][None, :]).astype(jnp.bfloat16)
    wr2 = _bdiag(wr).astype(jnp.bfloat16)
    wz2 = _bdiag(wz).astype(jnp.bfloat16)
    wn2 = _bdiag(wn).astype(jnp.bfloat16)
    two = lambda b: jnp.concatenate([b, b])[None, :]
    br = two(b_ih[0:D] + b_hh[0:D])
    bz = two(b_ih[D:2 * D] + b_hh[D:2 * D])
    bin_ = two(b_ih[2 * D:3 * D])
    bhn = two(b_hh[2 * D:3 * D])
    w1t2 = _bdiag(w1t)
    x_f = x_pad.reshape(NPAD // 2, 2 * D)
    dinv_f = jnp.broadcast_to(dinv.reshape(NPAD // 2, 2, 1),
                              (NPAD // 2, 2, D)).reshape(NPAD // 2, 2 * D)
    full = lambda s: pl.BlockSpec(s, lambda i: (0,) * len(s))
    y_f = pl.pallas_call(
        _gru_body,
        out_shape=jax.ShapeDtypeStruct((NPAD // 2, 2 * D), jnp.float32),
        grid=(NPAD // MBLK,),
        in_specs=[
            pl.BlockSpec((MF, 2 * D), lambda i: (i, 0)),
            full((2 * D, 2 * D)), full((2 * D, 2 * D)), full((2 * D, 2 * D)),
            full((2 * D, 2 * D)), full((2 * D, 2 * D)), full((2 * D, 2 * D)),
            full((1, 2 * D)), full((1, 2 * D)), full((1, 2 * D)), full((1, 2 * D)),
            full((2 * D, 2 * D)),
            pl.BlockSpec((MF, 2 * D), lambda i: (i, 0)),
        ],
        out_specs=pl.BlockSpec((MF, 2 * D), lambda i: (i, 0)),
    )(x_f, wr2, wz2, wn2, mr, mz, mn, br, bz, bin_, bhn, w1t2, dinv_f)
    return y_f.reshape(NPAD, D)


def _mid_body(y_ref, p_ref, dinv_ref, b_ref, wt_ref, o_ref):
    agg = y_ref[...] + p_ref[0] + p_ref[1]
    h = jax.nn.relu(agg * dinv_ref[...] + b_ref[...])
    o_ref[...] = jnp.dot(h, wt_ref[...], preferred_element_type=jnp.float32) * dinv_ref[...]


def _final_body(y_ref, p_ref, dinv_ref, b_ref, o_ref):
    agg = y_ref[...] + p_ref[0] + p_ref[1]
    o_ref[...] = jax.nn.relu(agg * dinv_ref[...] + b_ref[...])


def _run_mid(y, parts, dinv, b, wt):
    full = lambda s: pl.BlockSpec(s, lambda i: (0,) * len(s))
    return pl.pallas_call(
        _mid_body,
        out_shape=jax.ShapeDtypeStruct((NPAD, D), jnp.float32),
        grid=(NPAD // MBLK,),
        in_specs=[
            pl.BlockSpec((MBLK, D), lambda i: (i, 0)),
            pl.BlockSpec((NC, MBLK, D), lambda i: (0, i, 0)),
            pl.BlockSpec((MBLK, 1), lambda i: (i, 0)),
            full((1, D)), full((D, D)),
        ],
        out_specs=pl.BlockSpec((MBLK, D), lambda i: (i, 0)),
    )(y, parts, dinv, b, wt)


def _run_final(y, parts, dinv, b):
    full = lambda s: pl.BlockSpec(s, lambda i: (0,) * len(s))
    return pl.pallas_call(
        _final_body,
        out_shape=jax.ShapeDtypeStruct((NPAD, D), jnp.float32),
        grid=(NPAD // MBLK,),
        in_specs=[
            pl.BlockSpec((MBLK, D), lambda i: (i, 0)),
            pl.BlockSpec((NC, MBLK, D), lambda i: (0, i, 0)),
            pl.BlockSpec((MBLK, 1), lambda i: (i, 0)),
            full((1, D)),
        ],
        out_specs=pl.BlockSpec((MBLK, D), lambda i: (i, 0)),
    )(y, parts, dinv, b)


def _pool_body(pp_ref, cnt_ref, fcwt_ref, fcb_ref, o_ref):
    s = pp_ref[0, 0:G, :] + pp_ref[1, 0:G, :]
    pooled = s / jnp.maximum(cnt_ref[...], 1.0)
    o_ref[...] = jnp.dot(pooled, fcwt_ref[...], preferred_element_type=jnp.float32) + fcb_ref[...]


def _run_pool(pool_parts, cnt, fc_w, fc_b):
    fcwt = jnp.zeros((D, 128), jnp.float32).at[:, 0:10].set(jnp.transpose(fc_w))
    fcb = jnp.zeros((1, 128), jnp.float32).at[0, 0:10].set(fc_b)
    out = pl.pallas_call(
        _pool_body,
        out_shape=jax.ShapeDtypeStruct((G, 128), jnp.float32),
    )(pool_parts, cnt, fcwt, fcb)
    return out[:, 0:10]


def kernel(x, edge_index, batch, w_ih, w_hh, b_ih, b_hh, w1, b1, w2, b2,
           fc_w, fc_b):
    epad_junk = N + (jnp.arange(EPAD - E, dtype=jnp.int32) % (NPAD - N))
    src = jnp.concatenate([edge_index[0], epad_junk])
    dst = jnp.concatenate([edge_index[1], epad_junk])
    src3 = src.reshape(NW, KE, CHUNK)
    dst3 = dst.reshape(NW, KE, CHUNK)
    ppad_junk = jnp.arange(PPAD - N, dtype=jnp.int32)
    psrc = jnp.concatenate([jnp.arange(N, dtype=jnp.int32),
                            N + (ppad_junk % (NPAD - N))]).reshape(NW, KP, CHUNK)
    pdst = jnp.concatenate([batch, G + (ppad_junk % (GPAD - G))]).reshape(NW, KP, CHUNK)

    deg_p, cnt_p = _sc_counts(dst3, pdst)
    deg = deg_p[0] + deg_p[1] + 1.0
    dinv = lax.rsqrt(deg)[:, None]
    cnt = (cnt_p[0] + cnt_p[1])[0:G, None]

    x_pad = jnp.zeros((NPAD, T), jnp.float32).at[0:N].set(x)
    w1t = jnp.transpose(w1)
    y1 = _run_gru(x_pad, w_hh, w_ih, b_ih, b_hh, w1t, dinv)

    parts1 = _sc_scatter_edges(y1, src3, dst3)
    y2 = _run_mid(y1, parts1, dinv, b1[None, :], jnp.transpose(w2))

    parts2 = _sc_scatter_edges(y2, src3, dst3)
    h2 = _run_final(y2, parts2, dinv, b2[None, :])

    pool_parts = _sc_scatter_pool(h2, psrc, pdst)
    return _run_pool(pool_parts, cnt, fc_w, fc_b)
